# baseline (speedup 1.0000x reference)
import jax
import jax.numpy as jnp
from jax import lax
from jax.experimental import pallas as pl
from jax.experimental.pallas import tpu as pltpu
from jax.experimental.pallas import tpu_sc as plsc

B = 16384
C = 1000
NC = 2
NS = 16
NW = NC * NS
PER_W = B // NW
CHUNK = 128
NCHUNK = PER_W // CHUNK
L = 16


def _body(flat_logits, y3, out3, idx_v, out_v, sem):
    wid = lax.axis_index("s") * NC + lax.axis_index("c")
    base = wid * PER_W

    pltpu.sync_copy(y3.at[wid], idx_v)

    lanes = lax.iota(jnp.int32, L) * C
    for j in range(NCHUNK):
        for c in range(CHUNK // L):
            off = (base + j * CHUNK + c * L) * C
            seg = idx_v[j, pl.ds(c * L, L)]
            idx_v[j, pl.ds(c * L, L)] = seg + lanes + off

    copies = [
        pltpu.async_copy(flat_logits.at[idx_v.at[j]], out_v.at[j], sem)
        for j in range(NCHUNK)
    ]
    for cp in copies:
        cp.wait()

    pltpu.sync_copy(out_v, out3.at[wid])


@jax.jit
def kernel(logits, y):
    flat_logits = logits.reshape(-1)
    y3 = y.astype(jnp.int32).reshape(NW, NCHUNK, CHUNK)

    mesh = plsc.VectorSubcoreMesh(core_axis_name="c", subcore_axis_name="s")
    out3 = pl.kernel(
        _body,
        out_type=jax.ShapeDtypeStruct((NW, NCHUNK, CHUNK), jnp.float32),
        mesh=mesh,
        scratch_types=[
            pltpu.VMEM((NCHUNK, CHUNK), jnp.int32),
            pltpu.VMEM((NCHUNK, CHUNK), jnp.float32),
            pltpu.SemaphoreType.DMA,
        ],
    )(flat_logits, y3)
    return out3.reshape(-1)


# speedup vs baseline: 1.3649x; 1.3649x over previous
import jax
import jax.numpy as jnp
from jax import lax
from jax.experimental import pallas as pl
from jax.experimental.pallas import tpu as pltpu
from jax.experimental.pallas import tpu_sc as plsc

B = 16384
C = 1000
NC = 2
NS = 16
NW = NC * NS
PW = B // NW
NT = 8
L = 16
NV = PW // L
CAP = PW + NT * L + L
LASTCOL = C - 128


def _body(logits, last128, y3, out3, yv, tbl, posrec, res, outv, sem):
    wid = lax.axis_index("s") * NC + lax.axis_index("c")
    base = wid * PW

    pltpu.sync_copy(y3.at[wid], yv)

    iota = lax.iota(jnp.int32, L)
    zeros = jnp.zeros((L,), jnp.int32)

    cnt_vec = [zeros] * NT
    for v in range(NV):
        yvec = yv[v // 8, pl.ds((v % 8) * L, L)]
        t = lax.shift_right_logical(yvec, 7)
        for tt in range(NT):
            cnt_vec[tt] = cnt_vec[tt] + jnp.where(t == tt, 1, 0).astype(jnp.int32)

    cnt = [jnp.sum(cnt_vec[tt]) for tt in range(NT)]
    bases = []
    b = jnp.int32(0)
    for tt in range(NT):
        bases.append(b)
        b = b + ((cnt[tt] + 15) & -16)

    for tt in range(NT):
        off = pl.multiple_of(bases[tt] + (cnt[tt] & -16), L)
        tbl[pl.ds(off, L)] = zeros

    fill = [jnp.broadcast_to(bases[tt], (L,)).astype(jnp.int32) for tt in range(NT)]
    last = jnp.full((L,), L - 1, jnp.int32)
    for v in range(NV):
        yvec = yv[v // 8, pl.ds((v % 8) * L, L)]
        t = lax.shift_right_logical(yvec, 7)
        rowid = iota + (base + v * L)
        pos_acc = zeros
        for tt in range(NT):
            m = t == tt
            cs = plsc.cumsum(jnp.where(m, 1, 0).astype(jnp.int32))
            pos = fill[tt] + cs - 1
            plsc.store_scatter(tbl, [pos], rowid, mask=m)
            pos_acc = pos_acc + jnp.where(m, pos, 0)
            fill[tt] = fill[tt] + cs.at[last].get(mode="promise_in_bounds")
        posrec[v // 8, pl.ds((v % 8) * L, L)] = pos_acc

    total = jnp.int32(0)
    for tt in range(NT):
        trips = lax.shift_right_logical((cnt[tt] + 15) & -16, 4)
        bt = bases[tt]
        src = last128 if tt == NT - 1 else logits
        col = 0 if tt == NT - 1 else 128 * tt

        def _fire(k, carry, src=src, col=col, bt=bt):
            s = pl.multiple_of(bt + k * L, L)
            pltpu.async_copy(
                src.at[tbl.at[pl.ds(s, L)], pl.ds(col, 128)],
                res.at[pl.ds(s, L), :],
                sem,
            )
            return carry

        lax.fori_loop(0, trips, _fire, 0, unroll=False)
        total = total + trips

    def _drain(k, carry):
        pltpu.make_async_copy(
            logits.at[tbl.at[pl.ds(0, L)], pl.ds(0, 128)],
            res.at[pl.ds(0, L), :],
            sem,
        ).wait()
        return carry

    lax.fori_loop(0, total, _drain, 0, unroll=False)

    for v in range(NV):
        yvec = yv[v // 8, pl.ds((v % 8) * L, L)]
        t = lax.shift_right_logical(yvec, 7)
        colbase = jnp.where(t == NT - 1, LASTCOL, t * 128)
        lane = yvec - colbase
        pr = posrec[v // 8, pl.ds((v % 8) * L, L)]
        outv[v // 8, pl.ds((v % 8) * L, L)] = plsc.load_gather(res, [pr, lane])

    pltpu.sync_copy(outv, out3.at[wid])


@jax.jit
def kernel(logits, y):
    last128 = lax.slice(logits, (0, LASTCOL), (B, C))
    y3 = y.astype(jnp.int32).reshape(NW, PW // 128, 128)

    mesh = plsc.VectorSubcoreMesh(core_axis_name="c", subcore_axis_name="s")
    out3 = pl.kernel(
        _body,
        out_type=jax.ShapeDtypeStruct((NW, PW // 128, 128), jnp.float32),
        mesh=mesh,
        compiler_params=pltpu.CompilerParams(needs_layout_passes=False),
        scratch_types=[
            pltpu.VMEM((PW // 128, 128), jnp.int32),
            pltpu.VMEM((CAP,), jnp.int32),
            pltpu.VMEM((PW // 128, 128), jnp.int32),
            pltpu.VMEM((CAP, 128), jnp.float32),
            pltpu.VMEM((PW // 128, 128), jnp.float32),
            pltpu.SemaphoreType.DMA,
        ],
    )(logits, last128, y3)
    return out3.reshape(-1)


# speedup vs baseline: 5.2828x; 3.8705x over previous
import jax
import jax.numpy as jnp
from jax import lax
from jax.experimental import pallas as pl
from jax.experimental.pallas import tpu as pltpu
from jax.experimental.pallas import tpu_sc as plsc

B = 16384
C = 1000
NC = 2
NS = 16
NW = NC * NS
PW = B // NW
NB = PW // 128
L = 16


def _body(lt, y3, out3, yv, res, outv, sem):
    wid = lax.axis_index("s") * NC + lax.axis_index("c")

    pltpu.sync_copy(y3.at[wid], yv)

    copies = []
    for c in range(NB):
        col = pl.multiple_of((wid * NB + c) * 128, 128)
        copies.append(
            pltpu.async_copy(
                lt.at[yv.at[c], pl.ds(col, 128)], res.at[c], sem
            )
        )
    for cp in copies:
        cp.wait()

    iota = lax.iota(jnp.int32, L)
    for c in range(NB):
        for k in range(128 // L):
            d = iota + k * L
            outv[c, pl.ds(k * L, L)] = plsc.load_gather(res.at[c], [d, d])

    pltpu.sync_copy(outv, out3.at[wid])


@jax.jit
def kernel(logits, y):
    lt = logits.T
    y3 = y.astype(jnp.int32).reshape(NW, NB, 128)

    mesh = plsc.VectorSubcoreMesh(core_axis_name="c", subcore_axis_name="s")
    out3 = pl.kernel(
        _body,
        out_type=jax.ShapeDtypeStruct((NW, NB, 128), jnp.float32),
        mesh=mesh,
        compiler_params=pltpu.CompilerParams(needs_layout_passes=False),
        scratch_types=[
            pltpu.VMEM((NB, 128), jnp.int32),
            pltpu.VMEM((NB, 128, 128), jnp.float32),
            pltpu.VMEM((NB, 128), jnp.float32),
            pltpu.SemaphoreType.DMA,
        ],
    )(lt, y3)
    return out3.reshape(-1)


# speedup vs baseline: 6.9761x; 1.3205x over previous
import jax
import jax.numpy as jnp
from jax import lax
from jax.experimental import pallas as pl
from jax.experimental.pallas import tpu as pltpu
from jax.experimental.pallas import tpu_sc as plsc

B = 16384
C = 1000
NC = 2
NS = 16
NW = NC * NS
PW = B // NW
NB = PW // 128
L = 16


def _body(flat, y3, out3, yv, res, sem):
    wid = lax.axis_index("s") * NC + lax.axis_index("c")

    pltpu.sync_copy(y3.at[wid], yv)

    iota = lax.iota(jnp.int32, L)
    for c in range(NB):
        blk = (wid * NB + c) * 1024
        for k in range(128 // L):
            yvec = yv[c, pl.ds(k * L, L)]
            idx = (
                lax.shift_right_logical(yvec, 3) * 131072
                + (yvec & 7) * 128
                + (blk + k * L)
                + iota
            )
            yv[c, pl.ds(k * L, L)] = idx

    copies = [
        pltpu.async_copy(flat.at[yv.at[c]], res.at[c], sem) for c in range(NB)
    ]
    for cp in copies:
        cp.wait()

    pltpu.sync_copy(res, out3.at[wid])


@jax.jit
def kernel(logits, y):
    flat = (
        logits.T.reshape(C // 8, 8, B // 128, 128)
        .transpose(0, 2, 1, 3)
        .reshape(-1)
    )
    y3 = y.astype(jnp.int32).reshape(NW, NB, 128)

    mesh = plsc.VectorSubcoreMesh(core_axis_name="c", subcore_axis_name="s")
    out3 = pl.kernel(
        _body,
        out_type=jax.ShapeDtypeStruct((NW, NB, 128), jnp.float32),
        mesh=mesh,
        compiler_params=pltpu.CompilerParams(needs_layout_passes=False),
        scratch_types=[
            pltpu.VMEM((NB, 128), jnp.int32),
            pltpu.VMEM((NB, 128), jnp.float32),
            pltpu.SemaphoreType.DMA,
        ],
    )(flat, y3)
    return out3.reshape(-1)


# speedup vs baseline: 6.9770x; 1.0001x over previous
import jax
import jax.numpy as jnp
from jax import lax
from jax.experimental import pallas as pl
from jax.experimental.pallas import tpu as pltpu
from jax.experimental.pallas import tpu_sc as plsc

B = 16384
C = 1000
NC = 2
NS = 16
NW = NC * NS
PW = B // NW
NB = PW // 128
L = 16


def _body(flat, y3, out3, yv, res, sem):
    wid = lax.axis_index("s") * NC + lax.axis_index("c")

    pltpu.sync_copy(y3.at[wid], yv)

    iota = lax.iota(jnp.int32, L)
    copies = []
    for c in range(NB):
        blk = (wid * NB + c) * 1024

        def _cvt(k, carry, c=c, blk=blk):
            s = pl.multiple_of(k * L, L)
            yvec = yv[c, pl.ds(s, L)]
            idx = (
                lax.shift_right_logical(yvec, 3) * 131072
                + (yvec & 7) * 128
                + (blk + k * L)
                + iota
            )
            yv[c, pl.ds(s, L)] = idx
            return carry

        lax.fori_loop(0, 128 // L, _cvt, 0, unroll=False)
        copies.append(pltpu.async_copy(flat.at[yv.at[c]], res.at[c], sem))
    for cp in copies:
        cp.wait()

    pltpu.sync_copy(res, out3.at[wid])


@jax.jit
def kernel(logits, y):
    flat = (
        logits.T.reshape(C // 8, 8, B // 128, 128)
        .transpose(0, 2, 1, 3)
        .reshape(-1)
    )
    y3 = y.astype(jnp.int32).reshape(NW, NB, 128)

    mesh = plsc.VectorSubcoreMesh(core_axis_name="c", subcore_axis_name="s")
    out3 = pl.kernel(
        _body,
        out_type=jax.ShapeDtypeStruct((NW, NB, 128), jnp.float32),
        mesh=mesh,
        compiler_params=pltpu.CompilerParams(needs_layout_passes=False),
        scratch_types=[
            pltpu.VMEM((NB, 128), jnp.int32),
            pltpu.VMEM((NB, 128), jnp.float32),
            pltpu.SemaphoreType.DMA,
        ],
    )(flat, y3)
    return out3.reshape(-1)
